# baseline (speedup 1.0000x reference)
import jax
import jax.numpy as jnp
from jax import lax
from jax.experimental import pallas as pl
from jax.experimental.pallas import tpu as pltpu

_E = 64
_K = 8
_CH = 1024
_R = 3
_T = 16384
_HS = 4096
_NCH = _T // _CH


def _chunk_compute(xb, wf):
    logits = lax.dot_general(
        xb, wf, (((1,), (1,)), ((), ())),
        preferred_element_type=jnp.float32)
    m = jnp.max(logits, axis=-1, keepdims=True)
    unnorm = jnp.exp(logits - m)
    scores = unnorm / jnp.sum(unnorm, axis=-1, keepdims=True)

    iota = lax.broadcasted_iota(jnp.int32, scores.shape, 1)
    cur = scores
    ws = []
    idxs = []
    for _ in range(_K):
        mk = jnp.max(cur, axis=-1, keepdims=True)
        hit = cur == mk
        ik = jnp.min(jnp.where(hit, iota, _E), axis=-1, keepdims=True)
        ws.append(mk)
        idxs.append(ik)
        cur = jnp.where(iota == ik, -1.0, cur)
    return scores, jnp.concatenate(ws, axis=1), jnp.concatenate(idxs, axis=1)


def _router_body(x_hbm, w_ref, scores_hbm, topw_hbm, topi_hbm,
                 buf0, buf1, buf2, sbuf, wbuf, ibuf, insems, outsems):
    bufs = (buf0, buf1, buf2)
    wf = w_ref[...]

    def in_copy(i):
        return pltpu.make_async_copy(
            x_hbm.at[pl.ds(i * _CH, _CH), :], bufs[i % _R], insems.at[i % _R])

    def out_copies(i):
        s = i % 2
        sl = pl.ds(i * _CH, _CH)
        return (
            pltpu.make_async_copy(sbuf.at[s], scores_hbm.at[sl, :], outsems.at[s, 0]),
            pltpu.make_async_copy(wbuf.at[s], topw_hbm.at[sl, :], outsems.at[s, 1]),
            pltpu.make_async_copy(ibuf.at[s], topi_hbm.at[sl, :], outsems.at[s, 2]),
        )

    for i in range(_R - 1):
        in_copy(i).start()
    for i in range(_NCH):
        if i + _R - 1 < _NCH:
            in_copy(i + _R - 1).start()
        in_copy(i).wait()
        scores, topw, topi = _chunk_compute(bufs[i % _R][...], wf)
        s = i % 2
        if i >= 2:
            for c in out_copies(i - 2):
                c.wait()
        sbuf[s] = scores
        wbuf[s] = topw
        ibuf[s] = topi
        for c in out_copies(i):
            c.start()
    for i in (_NCH - 2, _NCH - 1):
        for c in out_copies(i):
            c.wait()


@jax.jit
def kernel(x, W):
    sl, bs, hs = x.shape
    t = sl * bs
    xt = x.reshape(t, hs)
    scores, topw, topi = pl.pallas_call(
        _router_body,
        in_specs=[
            pl.BlockSpec(memory_space=pl.ANY),
            pl.BlockSpec(memory_space=pltpu.VMEM),
        ],
        out_specs=[
            pl.BlockSpec(memory_space=pl.ANY),
            pl.BlockSpec(memory_space=pl.ANY),
            pl.BlockSpec(memory_space=pl.ANY),
        ],
        out_shape=[
            jax.ShapeDtypeStruct((t, _E), jnp.float32),
            jax.ShapeDtypeStruct((t, _K), jnp.float32),
            jax.ShapeDtypeStruct((t, _K), jnp.int32),
        ],
        scratch_shapes=[
            pltpu.VMEM((_CH, _HS), jnp.float32),
            pltpu.VMEM((_CH, _HS), jnp.float32),
            pltpu.VMEM((_CH, _HS), jnp.float32),
            pltpu.VMEM((2, _CH, _E), jnp.float32),
            pltpu.VMEM((2, _CH, _K), jnp.float32),
            pltpu.VMEM((2, _CH, _K), jnp.int32),
            pltpu.SemaphoreType.DMA((_R,)),
            pltpu.SemaphoreType.DMA((2, 3)),
        ],
    )(xt, W)
    return scores, topw, topi, jnp.float32(0.0)


# speedup vs baseline: 1.0458x; 1.0458x over previous
import jax
import jax.numpy as jnp
from jax import lax
from jax.experimental import pallas as pl
from jax.experimental.pallas import tpu as pltpu

_E = 64
_K = 8
_BT = 512


def _router_body(x_ref, w_ref, scores_ref, topw_ref, topi_ref):
    xb = x_ref[...]
    wf = w_ref[...]
    logits = lax.dot_general(
        xb, wf, (((1,), (1,)), ((), ())),
        preferred_element_type=jnp.float32)
    m = jnp.max(logits, axis=-1, keepdims=True)
    unnorm = jnp.exp(logits - m)
    scores = unnorm / jnp.sum(unnorm, axis=-1, keepdims=True)
    scores_ref[...] = scores

    iota = lax.broadcasted_iota(jnp.int32, scores.shape, 1)
    cur = scores
    ws = []
    idxs = []
    for _ in range(_K):
        mk = jnp.max(cur, axis=-1, keepdims=True)
        hit = cur == mk
        ik = jnp.min(jnp.where(hit, iota, _E), axis=-1, keepdims=True)
        ws.append(mk)
        idxs.append(ik)
        cur = jnp.where(iota == ik, -1.0, cur)
    topw_ref[...] = jnp.concatenate(ws, axis=1)
    topi_ref[...] = jnp.concatenate(idxs, axis=1)


@jax.jit
def kernel(x, W):
    sl, bs, hs = x.shape
    t = sl * bs
    xt = x.reshape(t, hs)
    grid = (t // _BT,)
    scores, topw, topi = pl.pallas_call(
        _router_body,
        grid=grid,
        in_specs=[
            pl.BlockSpec((_BT, hs), lambda i: (i, 0)),
            pl.BlockSpec(memory_space=pltpu.VMEM),
        ],
        out_specs=[
            pl.BlockSpec((_BT, _E), lambda i: (i, 0)),
            pl.BlockSpec((_BT, _K), lambda i: (i, 0)),
            pl.BlockSpec((_BT, _K), lambda i: (i, 0)),
        ],
        out_shape=[
            jax.ShapeDtypeStruct((t, _E), jnp.float32),
            jax.ShapeDtypeStruct((t, _K), jnp.float32),
            jax.ShapeDtypeStruct((t, _K), jnp.int32),
        ],
        compiler_params=pltpu.CompilerParams(
            dimension_semantics=("parallel",)),
    )(xt, W)
    return scores, topw, topi, jnp.float32(0.0)


# speedup vs baseline: 1.0851x; 1.0375x over previous
import jax
import jax.numpy as jnp
from jax import lax
from jax.experimental import pallas as pl
from jax.experimental.pallas import tpu as pltpu

_E = 64
_K = 8
_BT = 1024


def _router_body(x_ref, w_ref, scores_ref, topw_ref, topi_ref):
    xb = x_ref[...]
    wf = w_ref[...]
    logits = lax.dot_general(
        xb, wf, (((1,), (1,)), ((), ())),
        preferred_element_type=jnp.float32)
    m = jnp.max(logits, axis=-1, keepdims=True)
    unnorm = jnp.exp(logits - m)
    scores = unnorm / jnp.sum(unnorm, axis=-1, keepdims=True)
    scores_ref[...] = scores

    iota = lax.broadcasted_iota(jnp.int32, scores.shape, 1)
    cur = scores
    ws = []
    idxs = []
    for _ in range(_K):
        mk = jnp.max(cur, axis=-1, keepdims=True)
        hit = cur == mk
        ik = jnp.min(jnp.where(hit, iota, _E), axis=-1, keepdims=True)
        ws.append(mk)
        idxs.append(ik)
        cur = jnp.where(iota == ik, -1.0, cur)
    topw_ref[...] = jnp.concatenate(ws, axis=1)
    topi_ref[...] = jnp.concatenate(idxs, axis=1)


@jax.jit
def kernel(x, W):
    sl, bs, hs = x.shape
    t = sl * bs
    xt = x.reshape(t, hs)
    grid = (t // _BT,)
    scores, topw, topi = pl.pallas_call(
        _router_body,
        grid=grid,
        in_specs=[
            pl.BlockSpec((_BT, hs), lambda i: (i, 0)),
            pl.BlockSpec(memory_space=pltpu.VMEM),
        ],
        out_specs=[
            pl.BlockSpec((_BT, _E), lambda i: (i, 0)),
            pl.BlockSpec((_BT, _K), lambda i: (i, 0)),
            pl.BlockSpec((_BT, _K), lambda i: (i, 0)),
        ],
        out_shape=[
            jax.ShapeDtypeStruct((t, _E), jnp.float32),
            jax.ShapeDtypeStruct((t, _K), jnp.float32),
            jax.ShapeDtypeStruct((t, _K), jnp.int32),
        ],
        compiler_params=pltpu.CompilerParams(
            dimension_semantics=("arbitrary",)),
    )(xt, W)
    return scores, topw, topi, jnp.float32(0.0)
